# baseline (speedup 1.0000x reference)
import functools

import numpy as np

import jax
import jax.numpy as jnp
from jax import lax
from jax.experimental import pallas as pl
from jax.experimental.pallas import tpu as pltpu
from jax.experimental.pallas import tpu_sc as plsc

N = 10000
E = 320000
H = 128
G = 512

NC = 2
NS = 16
NW = NC * NS

NPAD = 10240
KGP = 80

SPLITS = (104960, 104960, 110080)

_mesh = lambda: plsc.VectorSubcoreMesh(core_axis_name="c", subcore_axis_name="s")

_PERM = np.zeros(H, dtype=np.int32)
for _t in range(H // 32):
    for _i in range(16):
        _PERM[32 * _t + 2 * _i] = 32 * _t + _i
        _PERM[32 * _t + 2 * _i + 1] = 32 * _t + 16 + _i


def _silu(v):
    return v * jax.nn.sigmoid(v)


RKG = 400
REPW = E // NW
RCH = REPW // RKG


def _radial_body(px_hbm, py_hbm, pz_hbm, row_hbm, col_hbm, out_hbm,
                 px, py, pz, ridx, cidx, radbuf):
    wid = lax.axis_index("s") * NC + lax.axis_index("c")
    base = wid * REPW
    pltpu.sync_copy(px_hbm, px)
    pltpu.sync_copy(py_hbm, py)
    pltpu.sync_copy(pz_hbm, pz)

    def chunk(i, carry):
        off = base + i * RKG
        pltpu.sync_copy(row_hbm.at[pl.ds(off, RKG)], ridx)
        pltpu.sync_copy(col_hbm.at[pl.ds(off, RKG)], cidx)

        def grp(j, c2):
            sl = pl.ds(j * 16, 16)
            ir = ridx[sl]
            ic = cidx[sl]
            dx = plsc.load_gather(px, [ir]) - plsc.load_gather(px, [ic])
            dy = plsc.load_gather(py, [ir]) - plsc.load_gather(py, [ic])
            dz = plsc.load_gather(pz, [ir]) - plsc.load_gather(pz, [ic])
            radbuf[sl] = dx * dx + dy * dy + dz * dz
            return c2

        lax.fori_loop(0, RKG // 16, grp, 0)
        pltpu.sync_copy(radbuf, out_hbm.at[pl.ds(off, RKG)])
        return carry

    lax.fori_loop(0, RCH, chunk, 0)


def _radial(px, py, pz, row, col):
    f = functools.partial(
        pl.kernel,
        out_type=jax.ShapeDtypeStruct((E,), jnp.float32),
        mesh=_mesh(),
        compiler_params=pltpu.CompilerParams(needs_layout_passes=False),
        scratch_types=[
            pltpu.VMEM((N,), jnp.float32),
            pltpu.VMEM((N,), jnp.float32),
            pltpu.VMEM((N,), jnp.float32),
            pltpu.VMEM((RKG,), jnp.int32),
            pltpu.VMEM((RKG,), jnp.int32),
            pltpu.VMEM((RKG,), jnp.float32),
        ],
    )(_radial_body)
    return f(px, py, pz, row, col)


def _tables_body(x_ref, wr_ref, wc_ref, b_ref, tr_ref, tc_ref):
    x = x_ref[...]
    tr_ref[...] = jnp.dot(x, wr_ref[...], preferred_element_type=jnp.float32) + b_ref[...]
    tc_ref[...] = jnp.dot(x, wc_ref[...], preferred_element_type=jnp.float32)


def _tables(x48, wr, wc, b):
    bn = 1000
    kdim = x48.shape[1]
    return pl.pallas_call(
        _tables_body,
        grid=(N // bn,),
        in_specs=[
            pl.BlockSpec((bn, kdim), lambda i: (i, 0)),
            pl.BlockSpec((kdim, H), lambda i: (0, 0)),
            pl.BlockSpec((kdim, H), lambda i: (0, 0)),
            pl.BlockSpec((1, H), lambda i: (0, 0)),
        ],
        out_specs=[
            pl.BlockSpec((bn, H), lambda i: (i, 0)),
            pl.BlockSpec((bn, H), lambda i: (i, 0)),
        ],
        out_shape=[
            jax.ShapeDtypeStruct((N, H), jnp.float32),
            jax.ShapeDtypeStruct((N, H), jnp.float32),
        ],
    )(x48, wr, wc, b)


def _gather_body(ne, tr_hbm, tc_hbm, row_hbm, col_hbm, rad_hbm, wrad_hbm,
                 out_hbm, ridx, cidx, radv, wradv,
                 bufr0, bufc0, bufr1, bufc1, pk0, pk1,
                 gr0, gc0, gr1, gc1, wb0, wb1):
    epw = ne // NW
    gch = epw // KGP
    wid = lax.axis_index("s") * NC + lax.axis_index("c")
    base = wid * epw
    pltpu.sync_copy(wrad_hbm, wradv)
    pltpu.sync_copy(row_hbm.at[pl.ds(base, epw)], ridx)
    pltpu.sync_copy(col_hbm.at[pl.ds(base, epw)], cidx)
    pltpu.sync_copy(rad_hbm.at[pl.ds(base, epw)], radv)
    wr = [wradv[pl.ds(t * 16, 16)] for t in range(H // 16)]
    slots = ((bufr0, bufc0, pk0, gr0, gc0, wb0),
             (bufr1, bufc1, pk1, gr1, gc1, wb1))

    def g_start(c, s):
        br, bc, _, sr, sc, _ = slots[s]
        loc = c * KGP
        pltpu.async_copy(tr_hbm.at[ridx.at[pl.ds(loc, KGP)]], br, sr)
        pltpu.async_copy(tc_hbm.at[cidx.at[pl.ds(loc, KGP)]], bc, sc)

    def g_wait(s):
        br, bc, _, sr, sc, _ = slots[s]
        pltpu.make_async_copy(tr_hbm.at[ridx.at[pl.ds(0, KGP)]], br, sr).wait()
        pltpu.make_async_copy(tc_hbm.at[cidx.at[pl.ds(0, KGP)]], bc, sc).wait()

    def add(c, s):
        br, bc, pk, _, _, _ = slots[s]

        def add_grp(g, c2):
            rv = radv[pl.ds(c * KGP + g * 16, 16)]
            for r in range(16):
                j = g * 16 + r
                rs = rv[r]
                for t in range(H // 32):
                    sa = pl.ds(32 * t, 16)
                    sb = pl.ds(32 * t + 16, 16)
                    a = br[j, sa] + bc[j, sa] + rs * wr[2 * t]
                    b = br[j, sb] + bc[j, sb] + rs * wr[2 * t + 1]
                    pk[j, pl.ds(16 * t, 16)] = plsc.bitcast(
                        plsc.pack(a, b, format=plsc.PackFormat.INTERLEAVED),
                        jnp.float32)
            return c2

        lax.fori_loop(0, KGP // 16, add_grp, 0)

    def wb_start(c, s):
        _, _, pk, _, _, sw = slots[s]
        pltpu.async_copy(pk, out_hbm.at[pl.ds(base + c * KGP, KGP)], sw)

    def wb_wait(s):
        _, _, pk, _, _, sw = slots[s]
        pltpu.make_async_copy(pk, out_hbm.at[pl.ds(0, KGP)], sw).wait()

    g_start(0, 0)
    g_wait(0)
    g_start(1, 1)
    add(0, 0)
    wb_start(0, 0)

    def pair(ii, carry):
        c = 2 * ii + 1
        g_wait(1)
        wb_wait(0)
        g_start(c + 1, 0)
        add(c, 1)
        wb_start(c, 1)
        g_wait(0)
        wb_wait(1)
        g_start(c + 2, 1)
        add(c + 1, 0)
        wb_start(c + 1, 0)
        return carry

    lax.fori_loop(0, (gch - 3) // 2, pair, 0)
    g_wait(1)
    wb_wait(0)
    g_start(gch - 1, 0)
    add(gch - 2, 1)
    wb_start(gch - 2, 1)
    g_wait(0)
    wb_wait(1)
    add(gch - 1, 0)
    wb_start(gch - 1, 0)
    wb_wait(0)


def _gather(tr, tc, row, col, rad, wrad):
    ne = row.shape[0]
    epw = ne // NW
    f = functools.partial(
        pl.kernel,
        out_type=jax.ShapeDtypeStruct((ne, H // 2), jnp.float32),
        mesh=_mesh(),
        compiler_params=pltpu.CompilerParams(needs_layout_passes=False),
        scratch_types=[
            pltpu.VMEM((epw,), jnp.int32),
            pltpu.VMEM((epw,), jnp.int32),
            pltpu.VMEM((epw,), jnp.float32),
            pltpu.VMEM((H,), jnp.float32),
            pltpu.VMEM((KGP, H), jnp.float32),
            pltpu.VMEM((KGP, H), jnp.float32),
            pltpu.VMEM((KGP, H), jnp.float32),
            pltpu.VMEM((KGP, H), jnp.float32),
            pltpu.VMEM((KGP, H // 2), jnp.float32),
            pltpu.VMEM((KGP, H // 2), jnp.float32),
            pltpu.SemaphoreType.DMA,
            pltpu.SemaphoreType.DMA,
            pltpu.SemaphoreType.DMA,
            pltpu.SemaphoreType.DMA,
            pltpu.SemaphoreType.DMA,
            pltpu.SemaphoreType.DMA,
        ],
    )(functools.partial(_gather_body, ne))
    return f(tr, tc, row, col, rad, wrad)


def _edge_body(zs_ref, f_ref, wf_ref, w1_ref, b1_ref, out_ref):
    z = (zs_ref[...].astype(jnp.float32)
         + jnp.dot(f_ref[...], wf_ref[...], preferred_element_type=jnp.float32))
    u = _silu(z)
    m = jnp.dot(u, w1_ref[...], preferred_element_type=jnp.float32) + b1_ref[...]
    out_ref[...] = _silu(m)


def _edge(zsum, feat, wf, w1, b1, be=1280):
    ne = zsum.shape[0]
    fd = feat.shape[1]
    return pl.pallas_call(
        _edge_body,
        grid=(ne // be,),
        in_specs=[
            pl.BlockSpec((be, H), lambda i: (i, 0)),
            pl.BlockSpec((be, fd), lambda i: (i, 0)),
            pl.BlockSpec((fd, H), lambda i: (0, 0)),
            pl.BlockSpec((H, H), lambda i: (0, 0)),
            pl.BlockSpec((1, H), lambda i: (0, 0)),
        ],
        out_specs=pl.BlockSpec((be, H), lambda i: (i, 0)),
        out_shape=jax.ShapeDtypeStruct((ne, H), jnp.float32),
    )(zsum, feat, wf, w1, b1)


def _scatter_body(ne, m_hbm, row3_hbm, zeros_hbm, out_hbm,
                  ridx2, buf0, buf1, acc,
                  ml0, ml1, sc0, sc1):
    epw = ne // NW
    sch = epw // KGP
    core = lax.axis_index("c")
    sub = lax.axis_index("s")
    wid = sub * NC + core
    rps = NPAD // NS
    base_e = wid * epw
    pltpu.sync_copy(zeros_hbm.at[pl.ds(sub * rps, rps)],
                    acc.at[pl.ds(sub * rps, rps)])
    pltpu.sync_copy(row3_hbm.at[wid], ridx2)
    plsc.subcore_barrier()

    slots = ((buf0, ml0, sc0), (buf1, ml1, sc1))

    def ml_start(c, s):
        buf, ml, _ = slots[s]
        pltpu.async_copy(m_hbm.at[pl.ds(base_e + c * KGP, KGP)], buf, ml)

    def ml_wait(s):
        buf, ml, _ = slots[s]
        pltpu.make_async_copy(m_hbm.at[pl.ds(0, KGP)], buf, ml).wait()

    def sc_start(c, s):
        buf, _, sc = slots[s]
        pltpu.async_copy(buf, acc.at[ridx2.at[c]], sc, add=True)

    def sc_wait(s):
        buf, _, sc = slots[s]
        pltpu.make_async_copy(buf, acc.at[ridx2.at[0]], sc).wait()

    ml_start(0, 0)
    ml_wait(0)
    ml_start(1, 1)
    sc_start(0, 0)

    def pair(ii, carry):
        c = 2 * ii + 1
        ml_wait(1)
        sc_wait(0)
        ml_start(c + 1, 0)
        sc_start(c, 1)
        ml_wait(0)
        sc_wait(1)
        ml_start(c + 2, 1)
        sc_start(c + 1, 0)
        return carry

    lax.fori_loop(0, (sch - 3) // 2, pair, 0)
    ml_wait(1)
    sc_wait(0)
    ml_start(sch - 1, 0)
    sc_start(sch - 2, 1)
    ml_wait(0)
    sc_wait(1)
    sc_start(sch - 1, 0)
    sc_wait(0)
    plsc.subcore_barrier()
    pltpu.sync_copy(acc.at[pl.ds(sub * rps, rps)],
                    out_hbm.at[core, pl.ds(sub * rps, rps)])


def _scatter(m, row3, zeros_pad):
    ne = m.shape[0]
    epw = ne // NW
    sch = epw // KGP
    f = functools.partial(
        pl.kernel,
        out_type=jax.ShapeDtypeStruct((NC, NPAD, H), jnp.float32),
        mesh=_mesh(),
        scratch_types=[
            pltpu.VMEM((sch, KGP), jnp.int32),
            pltpu.VMEM((KGP, H), jnp.float32),
            pltpu.VMEM((KGP, H), jnp.float32),
            pltpu.VMEM_SHARED((NPAD, H), jnp.float32),
            pltpu.SemaphoreType.DMA,
            pltpu.SemaphoreType.DMA,
            pltpu.SemaphoreType.DMA,
            pltpu.SemaphoreType.DMA,
        ],
    )(functools.partial(_scatter_body, ne))
    return f(m, row3, zeros_pad)


def _node0_body(x_ref, p0_ref, p1_ref, p2_ref, p3_ref, p4_ref, p5_ref,
                w0x_ref, w0a_ref, b0_ref, w1_ref, b1_ref,
                w1r_ref, w1c_ref, be1_ref,
                h1_ref, tr_ref, tc_ref):
    agg = (p0_ref[...] + p1_ref[...] + p2_ref[...]
           + p3_ref[...] + p4_ref[...] + p5_ref[...])
    t = _silu(jnp.dot(x_ref[...], w0x_ref[...], preferred_element_type=jnp.float32)
              + jnp.dot(agg, w0a_ref[...], preferred_element_type=jnp.float32)
              + b0_ref[...])
    h1 = jnp.dot(t, w1_ref[...], preferred_element_type=jnp.float32) + b1_ref[...]
    h1_ref[...] = h1
    tr_ref[...] = jnp.dot(h1, w1r_ref[...], preferred_element_type=jnp.float32) + be1_ref[...]
    tc_ref[...] = jnp.dot(h1, w1c_ref[...], preferred_element_type=jnp.float32)


def _node0(x48, parts, w0x, w0a, b0, w1, b1, w1r, w1c, be1):
    bn = 1000
    pspec = [pl.BlockSpec((bn, H), lambda i: (i, 0)) for _ in range(6)]
    return pl.pallas_call(
        _node0_body,
        grid=(N // bn,),
        in_specs=[pl.BlockSpec((bn, 48), lambda i: (i, 0))] + pspec + [
            pl.BlockSpec((48, H), lambda i: (0, 0)),
            pl.BlockSpec((H, H), lambda i: (0, 0)),
            pl.BlockSpec((1, H), lambda i: (0, 0)),
            pl.BlockSpec((H, H), lambda i: (0, 0)),
            pl.BlockSpec((1, H), lambda i: (0, 0)),
            pl.BlockSpec((H, H), lambda i: (0, 0)),
            pl.BlockSpec((H, H), lambda i: (0, 0)),
            pl.BlockSpec((1, H), lambda i: (0, 0)),
        ],
        out_specs=[
            pl.BlockSpec((bn, H), lambda i: (i, 0)),
            pl.BlockSpec((bn, H), lambda i: (i, 0)),
            pl.BlockSpec((bn, H), lambda i: (i, 0)),
        ],
        out_shape=[
            jax.ShapeDtypeStruct((N, H), jnp.float32),
            jax.ShapeDtypeStruct((N, H), jnp.float32),
            jax.ShapeDtypeStruct((N, H), jnp.float32),
        ],
    )(x48, *parts, w0x, w0a, b0, w1, b1, w1r, w1c, be1)


def _node1_body(h1_ref, p0_ref, p1_ref, p2_ref, p3_ref, p4_ref, p5_ref,
                w0h_ref, w0a_ref, b0_ref, w1_ref, b1_ref, h2_ref):
    h1 = h1_ref[...]
    agg = (p0_ref[...] + p1_ref[...] + p2_ref[...]
           + p3_ref[...] + p4_ref[...] + p5_ref[...])
    t = _silu(jnp.dot(h1, w0h_ref[...], preferred_element_type=jnp.float32)
              + jnp.dot(agg, w0a_ref[...], preferred_element_type=jnp.float32)
              + b0_ref[...])
    h2_ref[...] = h1 + jnp.dot(t, w1_ref[...], preferred_element_type=jnp.float32) + b1_ref[...]


def _node1(h1, parts, w0h, w0a, b0, w1, b1):
    bn = 1000
    pspec = [pl.BlockSpec((bn, H), lambda i: (i, 0)) for _ in range(6)]
    return pl.pallas_call(
        _node1_body,
        grid=(N // bn,),
        in_specs=[pl.BlockSpec((bn, H), lambda i: (i, 0))] + pspec + [
            pl.BlockSpec((H, H), lambda i: (0, 0)),
            pl.BlockSpec((H, H), lambda i: (0, 0)),
            pl.BlockSpec((1, H), lambda i: (0, 0)),
            pl.BlockSpec((H, H), lambda i: (0, 0)),
            pl.BlockSpec((1, H), lambda i: (0, 0)),
        ],
        out_specs=pl.BlockSpec((bn, H), lambda i: (i, 0)),
        out_shape=jax.ShapeDtypeStruct((N, H), jnp.float32),
    )(h1, *parts, w0h, w0a, b0, w1, b1)


def _pool_body(h_hbm, bidx_hbm, zeros_hbm, out_hbm, idx, buf, acc, sem):
    c = lax.axis_index("c")
    s = lax.axis_index("s")
    rps = NPAD // NS
    gps = G // NS

    @pl.when(c == 0)
    def _():
        pltpu.sync_copy(zeros_hbm.at[pl.ds(s * gps, gps)],
                        acc.at[pl.ds(s * gps, gps)])
        plsc.subcore_barrier()
        off = s * rps
        pltpu.sync_copy(bidx_hbm.at[pl.ds(off, rps)], idx)
        pltpu.sync_copy(h_hbm.at[pl.ds(off, rps)], buf)
        pltpu.sync_copy(buf, acc.at[idx], add=True)
        plsc.subcore_barrier()
        pltpu.sync_copy(acc.at[pl.ds(s * gps, gps)],
                        out_hbm.at[pl.ds(s * gps, gps)])


def _pool(h2pad, bidx_pad, zeros_pad):
    f = functools.partial(
        pl.kernel,
        out_type=jax.ShapeDtypeStruct((G, H), jnp.float32),
        mesh=_mesh(),
        scratch_types=[
            pltpu.VMEM((NPAD // NS,), jnp.int32),
            pltpu.VMEM((NPAD // NS, H), jnp.float32),
            pltpu.VMEM_SHARED((G, H), jnp.float32),
            pltpu.SemaphoreType.DMA,
        ],
    )(_pool_body)
    return f(h2pad, bidx_pad, zeros_pad)


def _as_bf16(zsum_packed):
    ne = zsum_packed.shape[0]
    return lax.bitcast_convert_type(zsum_packed, jnp.bfloat16).reshape(ne, H)


def _pad2(a, cols):
    return jnp.concatenate(
        [a, jnp.zeros((a.shape[0], cols - a.shape[1]), a.dtype)], axis=1)


def _split(a):
    o = 0
    out = []
    for ne in SPLITS:
        out.append(a[o:o + ne])
        o += ne
    return out


def kernel(x, edge_index, pos, edge_attr, batch_index, params):
    row = edge_index[0]
    col = edge_index[1]
    rows = _split(row)
    cols = _split(col)
    row3s = [r.reshape(NW, r.shape[0] // NW // KGP, KGP) for r in rows]
    x48 = _pad2(x, 48)
    ea8 = _pad2(edge_attr, 8)
    ea8s = _split(ea8)
    px, py, pz = pos[:, 0], pos[:, 1], pos[:, 2]
    zeros_pad = jnp.zeros((NPAD, H), jnp.float32)
    bidx_pad = jnp.concatenate(
        [batch_index, jnp.zeros((NPAD - N,), batch_index.dtype)])

    p0, p1 = params['l0'], params['l1']
    r1 = lambda v: v.reshape(1, H)

    eW0 = p0['e_W0']
    zp3 = jnp.zeros((3, H), jnp.float32)
    wr0 = jnp.concatenate([eW0[0:45], zp3], axis=0)
    wc0 = jnp.concatenate([eW0[45:90], zp3], axis=0)
    wrad0 = eW0[90]
    wea0 = jnp.concatenate([eW0[91:96], zp3], axis=0)
    nW0 = p0['n_W0']
    n0x = jnp.concatenate([nW0[0:45], zp3], axis=0)
    n0a = nW0[45:173]

    eW0b = p1['e_W0']
    w1r = eW0b[0:128]
    w1c = eW0b[128:256]
    wrad1 = eW0b[256]
    wfc1 = eW0b[257:385]
    nW0b = p1['n_W0']
    n1h = nW0b[0:128]
    n1a = nW0b[128:256]

    rad = _radial(px, py, pz, row, col)
    rads = _split(rad)

    t0r, t0c = _tables(x48, wr0, wc0, r1(p0['e_b0']))
    wea0p = wea0[:, _PERM]
    ew10p = p0['e_W1'][_PERM, :]
    wfc1p = wfc1[:, _PERM]
    ew11p = p1['e_W1'][_PERM, :]
    m0s, parts0 = [], []
    for i in range(3):
        zs = _gather(t0r, t0c, rows[i], cols[i], rads[i], wrad0)
        m0 = _edge(_as_bf16(zs), ea8s[i], wea0p, ew10p, r1(p0['e_b1']))
        m0s.append(m0)
        parts0.append(_scatter(m0, row3s[i], zeros_pad))
    parts0 = [p[c] for p in parts0 for c in range(NC)]
    h1, t1r, t1c = _node0(x48, parts0, n0x, n0a,
                          r1(p0['n_b0']), p0['n_W1'], r1(p0['n_b1']),
                          w1r, w1c, r1(p1['e_b0']))

    parts1 = []
    for i in range(3):
        zs = _gather(t1r, t1c, rows[i], cols[i], rads[i], wrad1)
        m1 = _edge(_as_bf16(zs), m0s[i], wfc1p, ew11p, r1(p1['e_b1']))
        parts1.append(_scatter(m1, row3s[i], zeros_pad))
    parts1 = [p[c] for p in parts1 for c in range(NC)]
    h2 = _node1(h1, parts1, n1h, n1a,
                r1(p1['n_b0']), p1['n_W1'], r1(p1['n_b1']))

    h2pad = jnp.concatenate([h2, jnp.zeros((NPAD - N, H), jnp.float32)], axis=0)
    graph_features = _pool(h2pad, bidx_pad, zeros_pad)

    return (graph_features, h2, batch_index)


# speedup vs baseline: 2.6973x; 2.6973x over previous
import functools

import jax
import jax.numpy as jnp
from jax import lax
from jax.experimental import pallas as pl
from jax.experimental.pallas import tpu as pltpu
from jax.experimental.pallas import tpu_sc as plsc

N = 10000
E = 320000
H = 128
G = 512

NC = 2
NS = 16
NW = NC * NS

NPAD = 10240
KGP = 80

SPLITS = (104960, 104960, 110080)

_mesh = lambda: plsc.VectorSubcoreMesh(core_axis_name="c", subcore_axis_name="s")


def _silu(v):
    return v * jax.nn.sigmoid(v)


RKG = 400
REPW = E // NW
RCH = REPW // RKG


def _radial_body(px_hbm, py_hbm, pz_hbm, row_hbm, col_hbm, out_hbm,
                 px, py, pz, ridx, cidx, radbuf):
    wid = lax.axis_index("s") * NC + lax.axis_index("c")
    base = wid * REPW
    pltpu.sync_copy(px_hbm, px)
    pltpu.sync_copy(py_hbm, py)
    pltpu.sync_copy(pz_hbm, pz)

    def chunk(i, carry):
        off = base + i * RKG
        pltpu.sync_copy(row_hbm.at[pl.ds(off, RKG)], ridx)
        pltpu.sync_copy(col_hbm.at[pl.ds(off, RKG)], cidx)

        def grp(j, c2):
            sl = pl.ds(j * 16, 16)
            ir = ridx[sl]
            ic = cidx[sl]
            dx = plsc.load_gather(px, [ir]) - plsc.load_gather(px, [ic])
            dy = plsc.load_gather(py, [ir]) - plsc.load_gather(py, [ic])
            dz = plsc.load_gather(pz, [ir]) - plsc.load_gather(pz, [ic])
            radbuf[sl] = dx * dx + dy * dy + dz * dz
            return c2

        lax.fori_loop(0, RKG // 16, grp, 0)
        pltpu.sync_copy(radbuf, out_hbm.at[pl.ds(off, RKG)])
        return carry

    lax.fori_loop(0, RCH, chunk, 0)


def _radial(px, py, pz, row, col):
    f = functools.partial(
        pl.kernel,
        out_type=jax.ShapeDtypeStruct((E,), jnp.float32),
        mesh=_mesh(),
        compiler_params=pltpu.CompilerParams(needs_layout_passes=False),
        scratch_types=[
            pltpu.VMEM((N,), jnp.float32),
            pltpu.VMEM((N,), jnp.float32),
            pltpu.VMEM((N,), jnp.float32),
            pltpu.VMEM((RKG,), jnp.int32),
            pltpu.VMEM((RKG,), jnp.int32),
            pltpu.VMEM((RKG,), jnp.float32),
        ],
    )(_radial_body)
    return f(px, py, pz, row, col)


def _tables_body(x_ref, wr_ref, wc_ref, b_ref, tr_ref, tc_ref):
    x = x_ref[...]
    tr_ref[...] = jnp.dot(x, wr_ref[...], preferred_element_type=jnp.float32) + b_ref[...]
    tc_ref[...] = jnp.dot(x, wc_ref[...], preferred_element_type=jnp.float32)


def _tables(x48, wr, wc, b):
    bn = 1000
    kdim = x48.shape[1]
    return pl.pallas_call(
        _tables_body,
        grid=(N // bn,),
        in_specs=[
            pl.BlockSpec((bn, kdim), lambda i: (i, 0)),
            pl.BlockSpec((kdim, H), lambda i: (0, 0)),
            pl.BlockSpec((kdim, H), lambda i: (0, 0)),
            pl.BlockSpec((1, H), lambda i: (0, 0)),
        ],
        out_specs=[
            pl.BlockSpec((bn, H), lambda i: (i, 0)),
            pl.BlockSpec((bn, H), lambda i: (i, 0)),
        ],
        out_shape=[
            jax.ShapeDtypeStruct((N, H), jnp.float32),
            jax.ShapeDtypeStruct((N, H), jnp.float32),
        ],
    )(x48, wr, wc, b)


def _gather_body(ne, tr_hbm, tc_hbm, row_hbm, col_hbm, rad_hbm, wrad_hbm,
                 out_hbm, ridx, cidx, radv, wradv,
                 bufr0, bufc0, bufr1, bufc1,
                 gr0, gc0, gr1, gc1, wb0, wb1):
    epw = ne // NW
    gch = epw // KGP
    wid = lax.axis_index("s") * NC + lax.axis_index("c")
    base = wid * epw
    pltpu.sync_copy(wrad_hbm, wradv)
    pltpu.sync_copy(row_hbm.at[pl.ds(base, epw)], ridx)
    pltpu.sync_copy(col_hbm.at[pl.ds(base, epw)], cidx)
    pltpu.sync_copy(rad_hbm.at[pl.ds(base, epw)], radv)
    wr = [wradv[pl.ds(t * 16, 16)] for t in range(H // 16)]
    slots = ((bufr0, bufc0, gr0, gc0, wb0), (bufr1, bufc1, gr1, gc1, wb1))

    def g_start(c, s):
        br, bc, sr, sc, _ = slots[s]
        loc = c * KGP
        pltpu.async_copy(tr_hbm.at[ridx.at[pl.ds(loc, KGP)]], br, sr)
        pltpu.async_copy(tc_hbm.at[cidx.at[pl.ds(loc, KGP)]], bc, sc)

    def g_wait(s):
        br, bc, sr, sc, _ = slots[s]
        pltpu.make_async_copy(tr_hbm.at[ridx.at[pl.ds(0, KGP)]], br, sr).wait()
        pltpu.make_async_copy(tc_hbm.at[cidx.at[pl.ds(0, KGP)]], bc, sc).wait()

    def add(c, s):
        br, bc, _, _, _ = slots[s]

        def add_grp(g, c2):
            rv = radv[pl.ds(c * KGP + g * 16, 16)]
            for r in range(16):
                rs = rv[r]
                for t in range(H // 16):
                    sl = pl.ds(t * 16, 16)
                    plsc.addupdate(br.at[g * 16 + r, sl],
                                   bc[g * 16 + r, sl] + rs * wr[t])
            return c2

        lax.fori_loop(0, KGP // 16, add_grp, 0)

    def wb_start(c, s):
        br, _, _, _, sw = slots[s]
        pltpu.async_copy(br, out_hbm.at[pl.ds(base + c * KGP, KGP)], sw)

    def wb_wait(s):
        br, _, _, _, sw = slots[s]
        pltpu.make_async_copy(br, out_hbm.at[pl.ds(0, KGP)], sw).wait()

    g_start(0, 0)
    g_wait(0)
    g_start(1, 1)
    add(0, 0)
    wb_start(0, 0)

    def pair(ii, carry):
        c = 2 * ii + 1
        g_wait(1)
        wb_wait(0)
        g_start(c + 1, 0)
        add(c, 1)
        wb_start(c, 1)
        g_wait(0)
        wb_wait(1)
        g_start(c + 2, 1)
        add(c + 1, 0)
        wb_start(c + 1, 0)
        return carry

    lax.fori_loop(0, (gch - 3) // 2, pair, 0)
    g_wait(1)
    wb_wait(0)
    g_start(gch - 1, 0)
    add(gch - 2, 1)
    wb_start(gch - 2, 1)
    g_wait(0)
    wb_wait(1)
    add(gch - 1, 0)
    wb_start(gch - 1, 0)
    wb_wait(0)


def _gather(tr, tc, row, col, rad, wrad):
    ne = row.shape[0]
    epw = ne // NW
    f = functools.partial(
        pl.kernel,
        out_type=jax.ShapeDtypeStruct((ne, H), jnp.float32),
        mesh=_mesh(),
        scratch_types=[
            pltpu.VMEM((epw,), jnp.int32),
            pltpu.VMEM((epw,), jnp.int32),
            pltpu.VMEM((epw,), jnp.float32),
            pltpu.VMEM((H,), jnp.float32),
            pltpu.VMEM((KGP, H), jnp.float32),
            pltpu.VMEM((KGP, H), jnp.float32),
            pltpu.VMEM((KGP, H), jnp.float32),
            pltpu.VMEM((KGP, H), jnp.float32),
            pltpu.SemaphoreType.DMA,
            pltpu.SemaphoreType.DMA,
            pltpu.SemaphoreType.DMA,
            pltpu.SemaphoreType.DMA,
            pltpu.SemaphoreType.DMA,
            pltpu.SemaphoreType.DMA,
        ],
    )(functools.partial(_gather_body, ne))
    return f(tr, tc, row, col, rad, wrad)


def _edge_body(zs_ref, f_ref, wf_ref, w1_ref, b1_ref, out_ref):
    z = zs_ref[...] + jnp.dot(f_ref[...], wf_ref[...],
                              preferred_element_type=jnp.float32)
    u = _silu(z)
    m = jnp.dot(u, w1_ref[...], preferred_element_type=jnp.float32) + b1_ref[...]
    out_ref[...] = _silu(m)


def _edge(zsum, feat, wf, w1, b1, be=1280):
    ne = zsum.shape[0]
    fd = feat.shape[1]
    return pl.pallas_call(
        _edge_body,
        grid=(ne // be,),
        in_specs=[
            pl.BlockSpec((be, H), lambda i: (i, 0)),
            pl.BlockSpec((be, fd), lambda i: (i, 0)),
            pl.BlockSpec((fd, H), lambda i: (0, 0)),
            pl.BlockSpec((H, H), lambda i: (0, 0)),
            pl.BlockSpec((1, H), lambda i: (0, 0)),
        ],
        out_specs=pl.BlockSpec((be, H), lambda i: (i, 0)),
        out_shape=jax.ShapeDtypeStruct((ne, H), jnp.float32),
    )(zsum, feat, wf, w1, b1)


def _scatter_body(ne, m_hbm, row3_hbm, zeros_hbm, out_hbm,
                  ridx2, buf0, buf1, acc,
                  ml0, ml1, sc0, sc1):
    epw = ne // NW
    sch = epw // KGP
    core = lax.axis_index("c")
    sub = lax.axis_index("s")
    wid = sub * NC + core
    rps = NPAD // NS
    base_e = wid * epw
    pltpu.sync_copy(zeros_hbm.at[pl.ds(sub * rps, rps)],
                    acc.at[pl.ds(sub * rps, rps)])
    pltpu.sync_copy(row3_hbm.at[wid], ridx2)
    plsc.subcore_barrier()

    slots = ((buf0, ml0, sc0), (buf1, ml1, sc1))

    def ml_start(c, s):
        buf, ml, _ = slots[s]
        pltpu.async_copy(m_hbm.at[pl.ds(base_e + c * KGP, KGP)], buf, ml)

    def ml_wait(s):
        buf, ml, _ = slots[s]
        pltpu.make_async_copy(m_hbm.at[pl.ds(0, KGP)], buf, ml).wait()

    def sc_start(c, s):
        buf, _, sc = slots[s]
        pltpu.async_copy(buf, acc.at[ridx2.at[c]], sc, add=True)

    def sc_wait(s):
        buf, _, sc = slots[s]
        pltpu.make_async_copy(buf, acc.at[ridx2.at[0]], sc).wait()

    ml_start(0, 0)
    ml_wait(0)
    ml_start(1, 1)
    sc_start(0, 0)

    def pair(ii, carry):
        c = 2 * ii + 1
        ml_wait(1)
        sc_wait(0)
        ml_start(c + 1, 0)
        sc_start(c, 1)
        ml_wait(0)
        sc_wait(1)
        ml_start(c + 2, 1)
        sc_start(c + 1, 0)
        return carry

    lax.fori_loop(0, (sch - 3) // 2, pair, 0)
    ml_wait(1)
    sc_wait(0)
    ml_start(sch - 1, 0)
    sc_start(sch - 2, 1)
    ml_wait(0)
    sc_wait(1)
    sc_start(sch - 1, 0)
    sc_wait(0)
    plsc.subcore_barrier()
    pltpu.sync_copy(acc.at[pl.ds(sub * rps, rps)],
                    out_hbm.at[core, pl.ds(sub * rps, rps)])


def _scatter(m, row3, zeros_pad):
    ne = m.shape[0]
    epw = ne // NW
    sch = epw // KGP
    f = functools.partial(
        pl.kernel,
        out_type=jax.ShapeDtypeStruct((NC, NPAD, H), jnp.float32),
        mesh=_mesh(),
        scratch_types=[
            pltpu.VMEM((sch, KGP), jnp.int32),
            pltpu.VMEM((KGP, H), jnp.float32),
            pltpu.VMEM((KGP, H), jnp.float32),
            pltpu.VMEM_SHARED((NPAD, H), jnp.float32),
            pltpu.SemaphoreType.DMA,
            pltpu.SemaphoreType.DMA,
            pltpu.SemaphoreType.DMA,
            pltpu.SemaphoreType.DMA,
        ],
    )(functools.partial(_scatter_body, ne))
    return f(m, row3, zeros_pad)


def _node0_body(x_ref, p0_ref, p1_ref, p2_ref, p3_ref, p4_ref, p5_ref,
                w0x_ref, w0a_ref, b0_ref, w1_ref, b1_ref,
                w1r_ref, w1c_ref, be1_ref,
                h1_ref, tr_ref, tc_ref):
    agg = (p0_ref[...] + p1_ref[...] + p2_ref[...]
           + p3_ref[...] + p4_ref[...] + p5_ref[...])
    t = _silu(jnp.dot(x_ref[...], w0x_ref[...], preferred_element_type=jnp.float32)
              + jnp.dot(agg, w0a_ref[...], preferred_element_type=jnp.float32)
              + b0_ref[...])
    h1 = jnp.dot(t, w1_ref[...], preferred_element_type=jnp.float32) + b1_ref[...]
    h1_ref[...] = h1
    tr_ref[...] = jnp.dot(h1, w1r_ref[...], preferred_element_type=jnp.float32) + be1_ref[...]
    tc_ref[...] = jnp.dot(h1, w1c_ref[...], preferred_element_type=jnp.float32)


def _node0(x48, parts, w0x, w0a, b0, w1, b1, w1r, w1c, be1):
    bn = 1000
    pspec = [pl.BlockSpec((bn, H), lambda i: (i, 0)) for _ in range(6)]
    return pl.pallas_call(
        _node0_body,
        grid=(N // bn,),
        in_specs=[pl.BlockSpec((bn, 48), lambda i: (i, 0))] + pspec + [
            pl.BlockSpec((48, H), lambda i: (0, 0)),
            pl.BlockSpec((H, H), lambda i: (0, 0)),
            pl.BlockSpec((1, H), lambda i: (0, 0)),
            pl.BlockSpec((H, H), lambda i: (0, 0)),
            pl.BlockSpec((1, H), lambda i: (0, 0)),
            pl.BlockSpec((H, H), lambda i: (0, 0)),
            pl.BlockSpec((H, H), lambda i: (0, 0)),
            pl.BlockSpec((1, H), lambda i: (0, 0)),
        ],
        out_specs=[
            pl.BlockSpec((bn, H), lambda i: (i, 0)),
            pl.BlockSpec((bn, H), lambda i: (i, 0)),
            pl.BlockSpec((bn, H), lambda i: (i, 0)),
        ],
        out_shape=[
            jax.ShapeDtypeStruct((N, H), jnp.float32),
            jax.ShapeDtypeStruct((N, H), jnp.float32),
            jax.ShapeDtypeStruct((N, H), jnp.float32),
        ],
    )(x48, *parts, w0x, w0a, b0, w1, b1, w1r, w1c, be1)


def _node1_body(h1_ref, p0_ref, p1_ref, p2_ref, p3_ref, p4_ref, p5_ref,
                w0h_ref, w0a_ref, b0_ref, w1_ref, b1_ref, h2_ref):
    h1 = h1_ref[...]
    agg = (p0_ref[...] + p1_ref[...] + p2_ref[...]
           + p3_ref[...] + p4_ref[...] + p5_ref[...])
    t = _silu(jnp.dot(h1, w0h_ref[...], preferred_element_type=jnp.float32)
              + jnp.dot(agg, w0a_ref[...], preferred_element_type=jnp.float32)
              + b0_ref[...])
    h2_ref[...] = h1 + jnp.dot(t, w1_ref[...], preferred_element_type=jnp.float32) + b1_ref[...]


def _node1(h1, parts, w0h, w0a, b0, w1, b1):
    bn = 1000
    pspec = [pl.BlockSpec((bn, H), lambda i: (i, 0)) for _ in range(6)]
    return pl.pallas_call(
        _node1_body,
        grid=(N // bn,),
        in_specs=[pl.BlockSpec((bn, H), lambda i: (i, 0))] + pspec + [
            pl.BlockSpec((H, H), lambda i: (0, 0)),
            pl.BlockSpec((H, H), lambda i: (0, 0)),
            pl.BlockSpec((1, H), lambda i: (0, 0)),
            pl.BlockSpec((H, H), lambda i: (0, 0)),
            pl.BlockSpec((1, H), lambda i: (0, 0)),
        ],
        out_specs=pl.BlockSpec((bn, H), lambda i: (i, 0)),
        out_shape=jax.ShapeDtypeStruct((N, H), jnp.float32),
    )(h1, *parts, w0h, w0a, b0, w1, b1)


def _pool_body(h_hbm, bidx_hbm, zeros_hbm, out_hbm, idx, buf, acc, sem):
    c = lax.axis_index("c")
    s = lax.axis_index("s")
    rps = NPAD // NS
    gps = G // NS

    @pl.when(c == 0)
    def _():
        pltpu.sync_copy(zeros_hbm.at[pl.ds(s * gps, gps)],
                        acc.at[pl.ds(s * gps, gps)])
        plsc.subcore_barrier()
        off = s * rps
        pltpu.sync_copy(bidx_hbm.at[pl.ds(off, rps)], idx)
        pltpu.sync_copy(h_hbm.at[pl.ds(off, rps)], buf)
        pltpu.sync_copy(buf, acc.at[idx], add=True)
        plsc.subcore_barrier()
        pltpu.sync_copy(acc.at[pl.ds(s * gps, gps)],
                        out_hbm.at[pl.ds(s * gps, gps)])


def _pool(h2pad, bidx_pad, zeros_pad):
    f = functools.partial(
        pl.kernel,
        out_type=jax.ShapeDtypeStruct((G, H), jnp.float32),
        mesh=_mesh(),
        scratch_types=[
            pltpu.VMEM((NPAD // NS,), jnp.int32),
            pltpu.VMEM((NPAD // NS, H), jnp.float32),
            pltpu.VMEM_SHARED((G, H), jnp.float32),
            pltpu.SemaphoreType.DMA,
        ],
    )(_pool_body)
    return f(h2pad, bidx_pad, zeros_pad)


def _pad2(a, cols):
    return jnp.concatenate(
        [a, jnp.zeros((a.shape[0], cols - a.shape[1]), a.dtype)], axis=1)


def _split(a):
    o = 0
    out = []
    for ne in SPLITS:
        out.append(a[o:o + ne])
        o += ne
    return out


def kernel(x, edge_index, pos, edge_attr, batch_index, params):
    row = edge_index[0]
    col = edge_index[1]
    rows = _split(row)
    cols = _split(col)
    row3s = [r.reshape(NW, r.shape[0] // NW // KGP, KGP) for r in rows]
    x48 = _pad2(x, 48)
    ea8 = _pad2(edge_attr, 8)
    ea8s = _split(ea8)
    px, py, pz = pos[:, 0], pos[:, 1], pos[:, 2]
    zeros_pad = jnp.zeros((NPAD, H), jnp.float32)
    bidx_pad = jnp.concatenate(
        [batch_index, jnp.zeros((NPAD - N,), batch_index.dtype)])

    p0, p1 = params['l0'], params['l1']
    r1 = lambda v: v.reshape(1, H)

    eW0 = p0['e_W0']
    zp3 = jnp.zeros((3, H), jnp.float32)
    wr0 = jnp.concatenate([eW0[0:45], zp3], axis=0)
    wc0 = jnp.concatenate([eW0[45:90], zp3], axis=0)
    wrad0 = eW0[90]
    wea0 = jnp.concatenate([eW0[91:96], zp3], axis=0)
    nW0 = p0['n_W0']
    n0x = jnp.concatenate([nW0[0:45], zp3], axis=0)
    n0a = nW0[45:173]

    eW0b = p1['e_W0']
    w1r = eW0b[0:128]
    w1c = eW0b[128:256]
    wrad1 = eW0b[256]
    wfc1 = eW0b[257:385]
    nW0b = p1['n_W0']
    n1h = nW0b[0:128]
    n1a = nW0b[128:256]

    rad = _radial(px, py, pz, row, col)
    rads = _split(rad)

    t0r, t0c = _tables(x48, wr0, wc0, r1(p0['e_b0']))
    m0s, parts0 = [], []
    for i in range(3):
        zs = _gather(t0r, t0c, rows[i], cols[i], rads[i], wrad0)
        m0 = _edge(zs, ea8s[i], wea0, p0['e_W1'], r1(p0['e_b1']))
        m0s.append(m0)
        parts0.append(_scatter(m0, row3s[i], zeros_pad))
    parts0 = [p[c] for p in parts0 for c in range(NC)]
    h1, t1r, t1c = _node0(x48, parts0, n0x, n0a,
                          r1(p0['n_b0']), p0['n_W1'], r1(p0['n_b1']),
                          w1r, w1c, r1(p1['e_b0']))

    parts1 = []
    for i in range(3):
        zs = _gather(t1r, t1c, rows[i], cols[i], rads[i], wrad1)
        m1 = _edge(zs, m0s[i], wfc1, p1['e_W1'], r1(p1['e_b1']))
        parts1.append(_scatter(m1, row3s[i], zeros_pad))
    parts1 = [p[c] for p in parts1 for c in range(NC)]
    h2 = _node1(h1, parts1, n1h, n1a,
                r1(p1['n_b0']), p1['n_W1'], r1(p1['n_b1']))

    h2pad = jnp.concatenate([h2, jnp.zeros((NPAD - N, H), jnp.float32)], axis=0)
    graph_features = _pool(h2pad, bidx_pad, zeros_pad)

    return (graph_features, h2, batch_index)
